# baseline (speedup 1.0000x reference)
import functools

import jax
import jax.numpy as jnp
from jax.experimental import pallas as pl


def _dn(a, b):
    return jax.lax.dot_general(a, b, (((1,), (1,)), ((), ())),
                               preferred_element_type=jnp.float32)


def _body(B, P, S, H, N,
          sse_ref, valid_ref, actx_ref, nctx_tab_ref, npos_tab_ref,
          lx_ref, ly_ref, Wm_ref, Wih_ref, Whh_ref,
          bm_ref, bi_ref, bh_ref, lctx_ref, out_ref, lctx_out_ref):
    lctx_out_ref[...] = lctx_ref[...]
    starts = sse_ref[:, 0:1]
    ends = sse_ref[:, 1:2]
    iota_n = jax.lax.broadcasted_iota(jnp.int32, (B, N), 1)
    onehot = (iota_n == starts).astype(jnp.float32)

    nctx = jnp.dot(onehot, nctx_tab_ref[...],
                   preferred_element_type=jnp.float32)
    nposg = jnp.dot(onehot, npos_tab_ref[...],
                    preferred_element_type=jnp.float32)
    npx = nposg[:, 0:1]
    npy = nposg[:, 1:2]

    actx = actx_ref[...]
    Wm = Wm_ref[...]
    xg = (_dn(nctx, Wm[:, 2:2 + H]) + _dn(actx, Wm[:, 2 + H:])
          + _dn(-nposg, Wm[:, 0:2]) + bm_ref[...])
    x = jnp.maximum(xg, 0.0)

    gi = _dn(x, Wih_ref[...]) + bi_ref[...]
    gh = _dn(nctx, Whh_ref[...]) + bh_ref[...]
    r_g = jax.nn.sigmoid(gi[:, :H] + gh[:, :H])
    z = jax.nn.sigmoid(gi[:, H:2 * H] + gh[:, H:2 * H])
    n_g = jnp.tanh(gi[:, 2 * H:] + r_g * gh[:, 2 * H:])
    r = (1.0 - z) * n_g + z * nctx

    cond = jnp.logical_and(valid_ref[...], (ends - starts) > 0)

    d2min = jnp.full((B, P), jnp.inf, jnp.float32)
    nan_any = jnp.zeros((B, P), jnp.bool_)
    for s in range(S):
        lxs = lx_ref[s]
        lys = ly_ref[s]
        nan_any = nan_any | jnp.isnan(lxs) | jnp.isnan(lys)
        dx = npx - lxs
        dy = npy - lys
        d2min = jnp.minimum(d2min, dx * dx + dy * dy)
    d2 = jnp.where(nan_any, npx * npx + npy * npy, d2min)
    keep = cond & (d2 < 10000.0)

    for l in range(P):
        out_ref[:, l, :] = jnp.where(keep[:, l:l + 1], r, 0.0)


def kernel(agent_pos, agent_context, ngh_pos, ngh_context, possible_lanes,
           lane_context, label, seq_start_end, valid_neighbor,
           W_msg, b_msg, W_ih, W_hh, b_ih, b_hh):
    B, P, H = lane_context.shape
    S = possible_lanes.shape[0]
    N = ngh_context.shape[0]

    lx = possible_lanes[:, :, 0].reshape(S, B, P)
    ly = possible_lanes[:, :, 1].reshape(S, B, P)
    valid_b = valid_neighbor.reshape(B, 1)

    body = functools.partial(_body, B, P, S, H, N)
    out2, out1 = pl.pallas_call(
        body,
        out_shape=(jax.ShapeDtypeStruct((B, P, H), jnp.float32),
                   jax.ShapeDtypeStruct((B, P, H), jnp.float32)),
    )(seq_start_end, valid_b, agent_context, ngh_context, ngh_pos,
      lx, ly, W_msg, W_ih, W_hh, b_msg, b_ih, b_hh, lane_context)

    return (out1, out2)


# speedup vs baseline: 1.1690x; 1.1690x over previous
import functools

import jax
import jax.numpy as jnp
from jax.experimental import pallas as pl


def _dn(a, b):
    return jax.lax.dot_general(a, b, (((1,), (1,)), ((), ())),
                               preferred_element_type=jnp.float32)


def _body(B, P, S, H, N,
          sse_ref, valid_ref, actx_ref, nctx_tab_ref, npos_tab_ref,
          lx_ref, ly_ref, Wm_ref, Wih_ref, Whh_ref,
          bm_ref, bi_ref, bh_ref, out_ref):
    starts = sse_ref[:, 0:1]
    ends = sse_ref[:, 1:2]
    iota_n = jax.lax.broadcasted_iota(jnp.int32, (B, N), 1)
    onehot = (iota_n == starts).astype(jnp.float32)

    nctx = jnp.dot(onehot, nctx_tab_ref[...],
                   preferred_element_type=jnp.float32)
    nposg = jnp.dot(onehot, npos_tab_ref[...],
                    preferred_element_type=jnp.float32)
    npx = nposg[:, 0:1]
    npy = nposg[:, 1:2]

    actx = actx_ref[...]
    Wm = Wm_ref[...]
    xg = (_dn(nctx, Wm[:, 2:2 + H]) + _dn(actx, Wm[:, 2 + H:])
          + _dn(-nposg, Wm[:, 0:2]) + bm_ref[...])
    x = jnp.maximum(xg, 0.0)

    gi = _dn(x, Wih_ref[...]) + bi_ref[...]
    gh = _dn(nctx, Whh_ref[...]) + bh_ref[...]
    r_g = jax.nn.sigmoid(gi[:, :H] + gh[:, :H])
    z = jax.nn.sigmoid(gi[:, H:2 * H] + gh[:, H:2 * H])
    n_g = jnp.tanh(gi[:, 2 * H:] + r_g * gh[:, 2 * H:])
    r = (1.0 - z) * n_g + z * nctx

    cond = jnp.logical_and(valid_ref[...], (ends - starts) > 0)

    d2min = jnp.full((B, P), jnp.inf, jnp.float32)
    nan_any = jnp.zeros((B, P), jnp.bool_)
    for s in range(S):
        lxs = lx_ref[s]
        lys = ly_ref[s]
        nan_any = nan_any | jnp.isnan(lxs) | jnp.isnan(lys)
        dx = npx - lxs
        dy = npy - lys
        d2min = jnp.minimum(d2min, dx * dx + dy * dy)
    d2 = jnp.where(nan_any, npx * npx + npy * npy, d2min)
    keep = cond & (d2 < 10000.0)

    for l in range(P):
        out_ref[:, l, :] = jnp.where(keep[:, l:l + 1], r, 0.0)


def kernel(agent_pos, agent_context, ngh_pos, ngh_context, possible_lanes,
           lane_context, label, seq_start_end, valid_neighbor,
           W_msg, b_msg, W_ih, W_hh, b_ih, b_hh):
    B, P, H = lane_context.shape
    S = possible_lanes.shape[0]
    N = ngh_context.shape[0]

    lx = possible_lanes[:, :, 0].reshape(S, B, P)
    ly = possible_lanes[:, :, 1].reshape(S, B, P)
    valid_b = valid_neighbor.reshape(B, 1)

    body = functools.partial(_body, B, P, S, H, N)
    out2 = pl.pallas_call(
        body,
        out_shape=jax.ShapeDtypeStruct((B, P, H), jnp.float32),
    )(seq_start_end, valid_b, agent_context, ngh_context, ngh_pos,
      lx, ly, W_msg, W_ih, W_hh, b_msg, b_ih, b_hh)

    return (lane_context, out2)
